# Initial kernel scaffold; baseline (speedup 1.0000x reference)
import jax
import jax.numpy as jnp
from jax.experimental import pallas as pl


def kernel(x_tensor, nodes, t, in_u, in_tau, out_u, out_tau, d, delta_emb, phi_in_W1, phi_in_b1, phi_in_W2, phi_in_b2, phi_out_W1, phi_out_b1, phi_out_W2, phi_out_b2, comb_W1, comb_b1, comb_W2, comb_b2, score_W1, score_b1, score_W2, score_b2):
    raise NotImplementedError("write your pallas kernel here")


# speedup vs baseline: 7.7506x; 7.7506x over previous
import functools

import jax
import jax.numpy as jnp
from jax import lax
from jax.experimental import pallas as pl
from jax.experimental.pallas import tpu as pltpu
from jax.experimental.pallas import tpu_sc as plsc

N, T, F = 10000, 32, 128
B, K = 4096, 16
W = 32
T_VAL = 31
DELTA_DIM, MSG_DIM, HIDDEN, H_DIM = 16, 64, 128, 64
NT = N * T
M = B * 3 * K
MQ = B * 3

NC, NS, L = 2, 16, 16
NW = NC * NS


def _precompute_body(x_ref, wi_ref, wo_ref, pi_ref, po_ref):
    x = x_ref[...]
    pi_ref[...] = jnp.dot(x, wi_ref[...], preferred_element_type=jnp.float32)
    po_ref[...] = jnp.dot(x, wo_ref[...], preferred_element_type=jnp.float32)


def _precompute(x2, w_in_x, w_out_x):
    RB = 3200
    grid = NT // RB
    return pl.pallas_call(
        _precompute_body,
        grid=(grid,),
        in_specs=[
            pl.BlockSpec((RB, F), lambda i: (i, 0)),
            pl.BlockSpec((F, MSG_DIM), lambda i: (0, 0)),
            pl.BlockSpec((F, MSG_DIM), lambda i: (0, 0)),
        ],
        out_specs=[
            pl.BlockSpec((RB, MSG_DIM), lambda i: (i, 0)),
            pl.BlockSpec((RB, MSG_DIM), lambda i: (i, 0)),
        ],
        out_shape=[jax.ShapeDtypeStruct((NT, MSG_DIM), jnp.float32)] * 2,
    )(x2, w_in_x, w_out_x)


def _make_gather(table_rows, D, num_rows):
    m_w = num_rows // NW
    CH = 128
    n_ch = m_w // CH
    assert m_w % CH == 0 and num_rows % NW == 0
    mesh = plsc.VectorSubcoreMesh(core_axis_name="c", subcore_axis_name="s")

    @functools.partial(
        pl.kernel,
        mesh=mesh,
        compiler_params=pltpu.CompilerParams(use_tc_tiling_on_sc=False),
        out_type=jax.ShapeDtypeStruct((num_rows, D), jnp.float32),
        scratch_types=[
            pltpu.VMEM((m_w,), jnp.int32),
            pltpu.VMEM((m_w,), jnp.int32),
            pltpu.VMEM((n_ch, CH), jnp.int32),
            pltpu.VMEM((CH, D), jnp.float32),
            pltpu.SemaphoreType.DMA,
        ],
    )
    def k(table_hbm, u_hbm, tau_hbm, out_hbm, u_v, tau_v, idx_v, rows_v, sem):
        wid = lax.axis_index("s") * NC + lax.axis_index("c")
        base = pl.multiple_of(wid * m_w, CH)
        pltpu.sync_copy(u_hbm.at[pl.ds(base, m_w)], u_v)
        pltpu.sync_copy(tau_hbm.at[pl.ds(base, m_w)], tau_v)

        def idx_body(j, carry):
            s = pl.ds(j * L, L)
            c = pl.ds((j % (CH // L)) * L, L)
            idx_v[j // (CH // L), c] = u_v[s] * T + tau_v[s]
            return carry

        lax.fori_loop(0, m_w // L, idx_body, 0)

        def gather_body(j, carry):
            cp = pltpu.make_async_copy(table_hbm.at[idx_v.at[j]], rows_v, sem)
            cp.start()
            cp.wait()
            pltpu.sync_copy(
                rows_v, out_hbm.at[pl.ds(pl.multiple_of(base + j * CH, CH), CH)])
            return carry

        lax.fori_loop(0, n_ch, gather_body, 0)

    return k


def _final_body(gin_ref, gout_ref, xvt_ref, tin_ref, taui_ref,
                tauo_ref, d_ref, demb_ref,
                w1ei_ref, b1i_ref, w2i_ref, b2i_ref,
                w1eo_ref, b1o_ref, w2o_ref, b2o_ref,
                c1x_ref, c1i_ref, c1o_ref, cb1_ref, c2_ref, cb2_ref,
                s1a_ref, s1b_ref, s1c_ref, s1d_ref, sb1_ref, s2_ref, sb2_ref,
                out_ref):
    rows = gin_ref.shape[0]
    bq = xvt_ref.shape[0]
    bb = d_ref.shape[0]

    def side(g_ref, t3_ref, tau3_ref, w1e_ref, b1_ref, w2_ref, b2_ref, sign):
        dtab = jnp.dot(demb_ref[...], w1e_ref[...],
                       preferred_element_type=jnp.float32) + b1_ref[...]
        delta = jnp.clip(sign * (t3_ref[...] - tau3_ref[...]), 0, W)
        io = lax.broadcasted_iota(jnp.int32, (rows, W + 1), 1)
        oh = (delta == io).astype(jnp.float32)
        e = jnp.dot(oh, dtab, preferred_element_type=jnp.float32)
        pre = jnp.maximum(g_ref[...] + e, 0.0)
        msum = jnp.sum(pre.reshape(bq, K, MSG_DIM), axis=1) * (1.0 / K)
        return jnp.dot(msum, w2_ref[...],
                       preferred_element_type=jnp.float32) + b2_ref[...]

    m_in = side(gin_ref, tin_ref, taui_ref, w1ei_ref, b1i_ref, w2i_ref,
                b2i_ref, 1)
    m_out = side(gout_ref, tin_ref, tauo_ref, w1eo_ref, b1o_ref, w2o_ref,
                 b2o_ref, -1)

    pre1 = (jnp.dot(xvt_ref[...], c1x_ref[...],
                    preferred_element_type=jnp.float32)
            + jnp.dot(m_in, c1i_ref[...], preferred_element_type=jnp.float32)
            + jnp.dot(m_out, c1o_ref[...], preferred_element_type=jnp.float32)
            + cb1_ref[...])
    h1 = jnp.maximum(pre1, 0.0)
    h = jnp.maximum(
        jnp.dot(h1, c2_ref[...], preferred_element_type=jnp.float32)
        + cb2_ref[...], 0.0)
    h3 = h.reshape(bb, 3, H_DIM)
    d_norm = jnp.clip(d_ref[...], 0, T_VAL).astype(jnp.float32) / float(T_VAL)
    sc_pre = (jnp.dot(h3[:, 0, :], s1a_ref[...],
                      preferred_element_type=jnp.float32)
              + jnp.dot(h3[:, 1, :], s1b_ref[...],
                        preferred_element_type=jnp.float32)
              + jnp.dot(h3[:, 2, :], s1c_ref[...],
                        preferred_element_type=jnp.float32)
              + d_norm * s1d_ref[...] + sb1_ref[...])
    logits = (jnp.dot(jnp.maximum(sc_pre, 0.0), s2_ref[...],
                      preferred_element_type=jnp.float32) + sb2_ref[...])
    out_ref[...] = logits


def _final(g_in, g_out, xvt, tin3, taui3, tauo3, d2, delta_emb,
           w1ei, b1i, w2i, b2i, w1eo, b1o, w2o, b2o,
           c1x, c1i, c1o, cb1, c2, cb2, s1a, s1b, s1c, s1d, sb1, s2, sb2):
    Bb = 128
    NB = B // Bb
    rows = Bb * 3 * K
    bq = Bb * 3

    def full(shape):
        return pl.BlockSpec(shape, lambda i: tuple(0 for _ in shape))

    return pl.pallas_call(
        _final_body,
        grid=(NB,),
        in_specs=[
            pl.BlockSpec((rows, MSG_DIM), lambda i: (i, 0)),
            pl.BlockSpec((rows, MSG_DIM), lambda i: (i, 0)),
            pl.BlockSpec((bq, F), lambda i: (i, 0)),
            pl.BlockSpec((rows, 1), lambda i: (i, 0)),
            pl.BlockSpec((rows, 1), lambda i: (i, 0)),
            pl.BlockSpec((rows, 1), lambda i: (i, 0)),
            pl.BlockSpec((Bb, 1), lambda i: (i, 0)),
            full((W + 1, DELTA_DIM)),
            full((DELTA_DIM, MSG_DIM)), full((MSG_DIM,)),
            full((MSG_DIM, MSG_DIM)), full((MSG_DIM,)),
            full((DELTA_DIM, MSG_DIM)), full((MSG_DIM,)),
            full((MSG_DIM, MSG_DIM)), full((MSG_DIM,)),
            full((F, HIDDEN)), full((MSG_DIM, HIDDEN)), full((MSG_DIM, HIDDEN)),
            full((HIDDEN,)),
            full((HIDDEN, H_DIM)), full((H_DIM,)),
            full((H_DIM, HIDDEN)), full((H_DIM, HIDDEN)), full((H_DIM, HIDDEN)),
            full((1, HIDDEN)), full((HIDDEN,)),
            full((HIDDEN, 1)), full((1,)),
        ],
        out_specs=pl.BlockSpec((Bb, 1), lambda i: (i, 0)),
        out_shape=jax.ShapeDtypeStruct((B, 1), jnp.float32),
    )(g_in, g_out, xvt, tin3, taui3, tauo3, d2, delta_emb,
      w1ei, b1i, w2i, b2i, w1eo, b1o, w2o, b2o,
      c1x, c1i, c1o, cb1, c2, cb2, s1a, s1b, s1c, s1d, sb1, s2, sb2)


def kernel(x_tensor, nodes, t, in_u, in_tau, out_u, out_tau, d, delta_emb,
           phi_in_W1, phi_in_b1, phi_in_W2, phi_in_b2,
           phi_out_W1, phi_out_b1, phi_out_W2, phi_out_b2,
           comb_W1, comb_b1, comb_W2, comb_b2,
           score_W1, score_b1, score_W2, score_b2):
    x2 = x_tensor.reshape(NT, F)
    pin, pout = _precompute(x2, phi_in_W1[:F], phi_out_W1[:F])

    iu = in_u.reshape(M).astype(jnp.int32)
    itau = in_tau.reshape(M).astype(jnp.int32)
    ou = out_u.reshape(M).astype(jnp.int32)
    otau = out_tau.reshape(M).astype(jnp.int32)
    qn = nodes.reshape(MQ).astype(jnp.int32)
    qt = jnp.broadcast_to(t[:, None], (B, 3)).reshape(MQ).astype(jnp.int32)

    g_in = _make_gather(NT, MSG_DIM, M)(pin, iu, itau)
    g_out = _make_gather(NT, MSG_DIM, M)(pout, ou, otau)
    xvt = _make_gather(NT, F, MQ)(x2, qn, qt)

    t3 = jnp.broadcast_to(t[:, None, None], (B, 3, K)).reshape(M, 1)
    t3 = t3.astype(jnp.int32)
    taui3 = in_tau.reshape(M, 1).astype(jnp.int32)
    tauo3 = out_tau.reshape(M, 1).astype(jnp.int32)
    d2 = d.reshape(B, 1).astype(jnp.int32)

    logits = _final(
        g_in, g_out, xvt, t3, taui3, tauo3, d2, delta_emb,
        phi_in_W1[F:], phi_in_b1, phi_in_W2, phi_in_b2,
        phi_out_W1[F:], phi_out_b1, phi_out_W2, phi_out_b2,
        comb_W1[:F], comb_W1[F:F + MSG_DIM], comb_W1[F + MSG_DIM:], comb_b1,
        comb_W2, comb_b2,
        score_W1[:H_DIM], score_W1[H_DIM:2 * H_DIM],
        score_W1[2 * H_DIM:3 * H_DIM], score_W1[3 * H_DIM:], score_b1,
        score_W2, score_b2)
    return logits.reshape(B)


# speedup vs baseline: 14.4514x; 1.8646x over previous
import functools

import jax
import jax.numpy as jnp
from jax import lax
from jax.experimental import pallas as pl
from jax.experimental.pallas import tpu as pltpu
from jax.experimental.pallas import tpu_sc as plsc

N, T, F = 10000, 32, 128
B, K = 4096, 16
W = 32
T_VAL = 31
DELTA_DIM, MSG_DIM, HIDDEN, H_DIM = 16, 64, 128, 64
NT = N * T
M = B * 3 * K
MQ = B * 3
MG = 2 * M + MQ

NC, NS, L = 2, 16, 16
NW = NC * NS


def _make_gather(D, num_rows):
    m_w = num_rows // NW
    CH = 128
    n_ch = m_w // CH
    assert m_w % CH == 0 and num_rows % NW == 0
    mesh = plsc.VectorSubcoreMesh(core_axis_name="c", subcore_axis_name="s")

    @functools.partial(
        pl.kernel,
        mesh=mesh,
        out_type=jax.ShapeDtypeStruct((num_rows, D), jnp.float32),
        scratch_types=[
            pltpu.VMEM((m_w,), jnp.int32),
            pltpu.VMEM((m_w,), jnp.int32),
            pltpu.VMEM((n_ch, CH), jnp.int32),
            pltpu.VMEM((2, CH, D), jnp.float32),
            pltpu.SemaphoreType.DMA,
            pltpu.SemaphoreType.DMA,
        ],
    )
    def k(table_hbm, u_hbm, tau_hbm, out_hbm, u_v, tau_v, idx_v, rows_v,
          sem0, sem1):
        sems = [sem0, sem1]
        wid = lax.axis_index("s") * NC + lax.axis_index("c")
        base = pl.multiple_of(wid * m_w, CH)
        pltpu.sync_copy(u_hbm.at[pl.ds(base, m_w)], u_v)
        pltpu.sync_copy(tau_hbm.at[pl.ds(base, m_w)], tau_v)

        def idx_body(j, carry):
            s = pl.ds(j * L, L)
            c = pl.ds((j % (CH // L)) * L, L)
            idx_v[j // (CH // L), c] = u_v[s] * T + tau_v[s]
            return carry

        lax.fori_loop(0, m_w // L, idx_body, 0)

        def start(j, p):
            pltpu.make_async_copy(
                table_hbm.at[idx_v.at[j]], rows_v.at[p], sems[0]).start()

        def finish(j, p):
            pltpu.make_async_copy(
                table_hbm.at[idx_v.at[j]], rows_v.at[p], sems[0]).wait()
            pltpu.sync_copy(
                rows_v.at[p],
                out_hbm.at[pl.ds(pl.multiple_of(base + j * CH, CH), CH)])

        start(0, 0)

        def gather_body(j, carry):
            start(j + 1, (j + 1) % 2)
            finish(j, j % 2)
            return carry

        lax.fori_loop(0, n_ch - 1, gather_body, 0)
        finish(n_ch - 1, (n_ch - 1) % 2)

    return k


def _final_body(gin_ref, gout_ref, xvt_ref, tin_ref, taui_ref,
                tauo_ref, d_ref, demb_ref,
                w1xi_ref, w1ei_ref, b1i_ref, w2i_ref, b2i_ref,
                w1xo_ref, w1eo_ref, b1o_ref, w2o_ref, b2o_ref,
                c1x_ref, c1i_ref, c1o_ref, cb1_ref, c2_ref, cb2_ref,
                s1a_ref, s1b_ref, s1c_ref, s1d_ref, sb1_ref, s2_ref, sb2_ref,
                out_ref):
    rows = gin_ref.shape[0]
    bq = xvt_ref.shape[0]
    bb = d_ref.shape[0]

    def side(g_ref, t3_ref, tau3_ref, w1x_ref, w1e_ref, b1_ref, w2_ref,
             b2_ref, sign):
        dtab = jnp.dot(demb_ref[...], w1e_ref[...],
                       preferred_element_type=jnp.float32) + b1_ref[...]
        delta = jnp.clip(sign * (t3_ref[...] - tau3_ref[...]), 0, W)
        io = lax.broadcasted_iota(jnp.int32, (rows, W + 1), 1)
        oh = (delta == io).astype(jnp.float32)
        e = jnp.dot(oh, dtab, preferred_element_type=jnp.float32)
        xw = jnp.dot(g_ref[...], w1x_ref[...],
                     preferred_element_type=jnp.float32)
        pre = jnp.maximum(xw + e, 0.0)
        msum = jnp.sum(pre.reshape(bq, K, MSG_DIM), axis=1) * (1.0 / K)
        return jnp.dot(msum, w2_ref[...],
                       preferred_element_type=jnp.float32) + b2_ref[...]

    m_in = side(gin_ref, tin_ref, taui_ref, w1xi_ref, w1ei_ref, b1i_ref,
                w2i_ref, b2i_ref, 1)
    m_out = side(gout_ref, tin_ref, tauo_ref, w1xo_ref, w1eo_ref, b1o_ref,
                 w2o_ref, b2o_ref, -1)

    pre1 = (jnp.dot(xvt_ref[...], c1x_ref[...],
                    preferred_element_type=jnp.float32)
            + jnp.dot(m_in, c1i_ref[...], preferred_element_type=jnp.float32)
            + jnp.dot(m_out, c1o_ref[...], preferred_element_type=jnp.float32)
            + cb1_ref[...])
    h1 = jnp.maximum(pre1, 0.0)
    h = jnp.maximum(
        jnp.dot(h1, c2_ref[...], preferred_element_type=jnp.float32)
        + cb2_ref[...], 0.0)
    h3 = h.reshape(bb, 3, H_DIM)
    d_norm = jnp.clip(d_ref[...], 0, T_VAL).astype(jnp.float32) / float(T_VAL)
    sc_pre = (jnp.dot(h3[:, 0, :], s1a_ref[...],
                      preferred_element_type=jnp.float32)
              + jnp.dot(h3[:, 1, :], s1b_ref[...],
                        preferred_element_type=jnp.float32)
              + jnp.dot(h3[:, 2, :], s1c_ref[...],
                        preferred_element_type=jnp.float32)
              + d_norm * s1d_ref[...] + sb1_ref[...])
    logits = (jnp.dot(jnp.maximum(sc_pre, 0.0), s2_ref[...],
                      preferred_element_type=jnp.float32) + sb2_ref[...])
    out_ref[...] = logits


def _final(g_all, tin3, taui3, tauo3, d2, delta_emb,
           w1xi, w1ei, b1i, w2i, b2i, w1xo, w1eo, b1o, w2o, b2o,
           c1x, c1i, c1o, cb1, c2, cb2, s1a, s1b, s1c, s1d, sb1, s2, sb2):
    Bb = 128
    NB = B // Bb
    rows = Bb * 3 * K
    bq = Bb * 3

    def full(shape):
        return pl.BlockSpec(shape, lambda i: tuple(0 for _ in shape))

    nin = M // rows
    nq = (2 * M) // bq

    return pl.pallas_call(
        _final_body,
        grid=(NB,),
        in_specs=[
            pl.BlockSpec((rows, F), lambda i: (i, 0)),
            pl.BlockSpec((rows, F), lambda i: (i + nin, 0)),
            pl.BlockSpec((bq, F), lambda i: (i + nq, 0)),
            pl.BlockSpec((rows, 1), lambda i: (i, 0)),
            pl.BlockSpec((rows, 1), lambda i: (i, 0)),
            pl.BlockSpec((rows, 1), lambda i: (i, 0)),
            pl.BlockSpec((Bb, 1), lambda i: (i, 0)),
            full((W + 1, DELTA_DIM)),
            full((F, MSG_DIM)), full((DELTA_DIM, MSG_DIM)), full((MSG_DIM,)),
            full((MSG_DIM, MSG_DIM)), full((MSG_DIM,)),
            full((F, MSG_DIM)), full((DELTA_DIM, MSG_DIM)), full((MSG_DIM,)),
            full((MSG_DIM, MSG_DIM)), full((MSG_DIM,)),
            full((F, HIDDEN)), full((MSG_DIM, HIDDEN)), full((MSG_DIM, HIDDEN)),
            full((HIDDEN,)),
            full((HIDDEN, H_DIM)), full((H_DIM,)),
            full((H_DIM, HIDDEN)), full((H_DIM, HIDDEN)), full((H_DIM, HIDDEN)),
            full((1, HIDDEN)), full((HIDDEN,)),
            full((HIDDEN, 1)), full((1,)),
        ],
        out_specs=pl.BlockSpec((Bb, 1), lambda i: (i, 0)),
        out_shape=jax.ShapeDtypeStruct((B, 1), jnp.float32),
    )(g_all, g_all, g_all, tin3, taui3, tauo3, d2, delta_emb,
      w1xi, w1ei, b1i, w2i, b2i, w1xo, w1eo, b1o, w2o, b2o,
      c1x, c1i, c1o, cb1, c2, cb2, s1a, s1b, s1c, s1d, sb1, s2, sb2)


def kernel(x_tensor, nodes, t, in_u, in_tau, out_u, out_tau, d, delta_emb,
           phi_in_W1, phi_in_b1, phi_in_W2, phi_in_b2,
           phi_out_W1, phi_out_b1, phi_out_W2, phi_out_b2,
           comb_W1, comb_b1, comb_W2, comb_b2,
           score_W1, score_b1, score_W2, score_b2):
    x2 = x_tensor.reshape(NT, F)

    qt = jnp.broadcast_to(t[:, None], (B, 3)).reshape(MQ)
    u_all = jnp.concatenate(
        [in_u.reshape(M), out_u.reshape(M), nodes.reshape(MQ)]
    ).astype(jnp.int32)
    tau_all = jnp.concatenate(
        [in_tau.reshape(M), out_tau.reshape(M), qt]).astype(jnp.int32)

    g_all = _make_gather(F, MG)(x2, u_all, tau_all)

    t3 = jnp.broadcast_to(t[:, None, None], (B, 3, K)).reshape(M, 1)
    t3 = t3.astype(jnp.int32)
    taui3 = in_tau.reshape(M, 1).astype(jnp.int32)
    tauo3 = out_tau.reshape(M, 1).astype(jnp.int32)
    d2 = d.reshape(B, 1).astype(jnp.int32)

    logits = _final(
        g_all, t3, taui3, tauo3, d2, delta_emb,
        phi_in_W1[:F], phi_in_W1[F:], phi_in_b1, phi_in_W2, phi_in_b2,
        phi_out_W1[:F], phi_out_W1[F:], phi_out_b1, phi_out_W2, phi_out_b2,
        comb_W1[:F], comb_W1[F:F + MSG_DIM], comb_W1[F + MSG_DIM:], comb_b1,
        comb_W2, comb_b2,
        score_W1[:H_DIM], score_W1[H_DIM:2 * H_DIM],
        score_W1[2 * H_DIM:3 * H_DIM], score_W1[3 * H_DIM:], score_b1,
        score_W2, score_b2)
    return logits.reshape(B)


# speedup vs baseline: 19.3250x; 1.3372x over previous
import functools

import jax
import jax.numpy as jnp
from jax import lax
from jax.experimental import pallas as pl
from jax.experimental.pallas import tpu as pltpu
from jax.experimental.pallas import tpu_sc as plsc

N, T, F = 10000, 32, 128
B, K = 4096, 16
W = 32
T_VAL = 31
DELTA_DIM, MSG_DIM, HIDDEN, H_DIM = 16, 64, 128, 64
NT = N * T
M = B * 3 * K
MQ = B * 3

NC, NS, L = 2, 16, 16
NW = NC * NS

QW = MQ // NW
CQ = 8
NCH = QW // CQ
CHA = CQ * K
AW = QW * K
NQCH = QW // 128


def _precompute_body(x_ref, w1cat_ref, demb_ref, w1ecat_ref, b1cat_ref,
                     px_ref, dtab_ref):
    px_ref[...] = jnp.dot(x_ref[...], w1cat_ref[...],
                          preferred_element_type=jnp.float32)
    dtab_ref[...] = jnp.dot(demb_ref[...], w1ecat_ref[...],
                            preferred_element_type=jnp.float32) + b1cat_ref[...]


def _precompute(x2, w1cat, demb, w1ecat, b1cat):
    RB = 3200
    grid = NT // RB

    def full(shape):
        return pl.BlockSpec(shape, lambda i: tuple(0 for _ in shape))

    return pl.pallas_call(
        _precompute_body,
        grid=(grid,),
        in_specs=[
            pl.BlockSpec((RB, F), lambda i: (i, 0)),
            full((F, 2 * MSG_DIM)),
            full((W + 1, DELTA_DIM)),
            full((DELTA_DIM, 2 * MSG_DIM)),
            full((2 * MSG_DIM,)),
        ],
        out_specs=[
            pl.BlockSpec((RB, 2 * MSG_DIM), lambda i: (i, 0)),
            full((W + 1, 2 * MSG_DIM)),
        ],
        out_shape=[
            jax.ShapeDtypeStruct((NT, 2 * MSG_DIM), jnp.float32),
            jax.ShapeDtypeStruct((W + 1, 2 * MSG_DIM), jnp.float32),
        ],
    )(x2, w1cat, demb, w1ecat, b1cat)


def _sc_messages():
    mesh = plsc.VectorSubcoreMesh(core_axis_name="c", subcore_axis_name="s")

    @functools.partial(
        pl.kernel,
        mesh=mesh,
        compiler_params=pltpu.CompilerParams(needs_layout_passes=False),
        out_type=[
            jax.ShapeDtypeStruct((MQ, 2 * MSG_DIM), jnp.float32),
            jax.ShapeDtypeStruct((MQ, F), jnp.float32),
        ],
        scratch_types=[
            pltpu.VMEM((AW,), jnp.int32),
            pltpu.VMEM((AW,), jnp.int32),
            pltpu.VMEM((QW,), jnp.int32),
            pltpu.VMEM((QW,), jnp.int32),
            pltpu.VMEM((NCH, CHA), jnp.int32),
            pltpu.VMEM((NCH, CHA), jnp.int32),
            pltpu.VMEM((NQCH, 128), jnp.int32),
            pltpu.VMEM((AW,), jnp.int32),
            pltpu.VMEM((AW,), jnp.int32),
            pltpu.VMEM((W + 1, 2 * MSG_DIM), jnp.float32),
            pltpu.VMEM((CHA, 2 * MSG_DIM), jnp.float32),
            pltpu.VMEM((CHA, 2 * MSG_DIM), jnp.float32),
            pltpu.VMEM((QW, 2 * MSG_DIM), jnp.float32),
            pltpu.SemaphoreType.DMA,
            pltpu.SemaphoreType.DMA,
        ],
    )
    def k(px_hbm, x2_hbm, dtab_hbm, iu_hbm, itau_hbm, ou_hbm, otau_hbm,
          qn_hbm, qt_hbm, msum_hbm, xvt_hbm,
          u_v, tau_v, qn_v, qt_v, idxi_v, idxo_v, idxq_v, deli_v, delo_v,
          dtab_v, rowsi_v, rowso_v, ost_v, semA, semB):
        wid = lax.axis_index("s") * NC + lax.axis_index("c")
        qbase = pl.multiple_of(wid * QW, 128)
        abase = pl.multiple_of(wid * AW, 128)
        iota = lax.iota(jnp.int32, L)

        pltpu.sync_copy(dtab_hbm, dtab_v)
        pltpu.sync_copy(qn_hbm.at[pl.ds(qbase, QW)], qn_v)
        pltpu.sync_copy(qt_hbm.at[pl.ds(qbase, QW)], qt_v)

        def qidx_body(g, carry):
            s = pl.ds(g * L, L)
            idxq_v[g // 8, pl.ds((g % 8) * L, L)] = qn_v[s] * T + qt_v[s]
            return carry

        lax.fori_loop(0, QW // L, qidx_body, 0)

        def prep_direction(uh, tauh, idx_v, del_v, sign):
            pltpu.sync_copy(uh.at[pl.ds(abase, AW)], u_v)
            pltpu.sync_copy(tauh.at[pl.ds(abase, AW)], tau_v)

            def body(g, carry):
                s = pl.ds(g * L, L)
                u16 = u_v[s]
                tau16 = tau_v[s]
                idx_v[g // 8, pl.ds((g % 8) * L, L)] = u16 * T + tau16
                tq = plsc.load_gather(qt_v, [iota * 0 + g])
                raw = (tq - tau16) * sign
                del_v[s] = jnp.minimum(jnp.maximum(raw, 0), W)
                return carry

            lax.fori_loop(0, AW // L, body, 0)

        prep_direction(iu_hbm, itau_hbm, idxi_v, deli_v, 1)
        prep_direction(ou_hbm, otau_hbm, idxo_v, delo_v, -1)

        def chunk_body(j, carry):
            cpi = pltpu.make_async_copy(px_hbm.at[idxi_v.at[j]], rowsi_v, semA)
            cpo = pltpu.make_async_copy(px_hbm.at[idxo_v.at[j]], rowso_v, semB)
            cpi.start()
            cpo.start()
            cpi.wait()
            cpo.wait()

            def q_body(qi, carry2):
                a0 = qi * K

                def reduce_dir(rows_v, del_v, colbase):
                    acc = [jnp.zeros((L,), jnp.float32) for _ in range(4)]
                    for kk in range(K):
                        a = a0 + kk
                        d16 = plsc.load_gather(
                            del_v, [iota * 0 + (j * CHA + a)])
                        for c in range(4):
                            dval = plsc.load_gather(
                                dtab_v, [d16, iota + (colbase + c * L)])
                            r = rows_v[a, pl.ds(colbase + c * L, L)]
                            acc[c] = acc[c] + jnp.maximum(r + dval, 0.0)
                    for c in range(4):
                        ost_v[j * CQ + qi,
                              pl.ds(colbase + c * L, L)] = acc[c]

                reduce_dir(rowsi_v, deli_v, 0)
                reduce_dir(rowso_v, delo_v, MSG_DIM)
                return carry2

            lax.fori_loop(0, CQ, q_body, 0)
            return carry

        lax.fori_loop(0, NCH, chunk_body, 0)
        pltpu.sync_copy(ost_v, msum_hbm.at[pl.ds(qbase, QW)])

        def q_gather_body(j, carry):
            cp = pltpu.make_async_copy(x2_hbm.at[idxq_v.at[j]],
                                       rowsi_v.at[pl.ds(0, 128)], semA)
            cp.start()
            cp.wait()
            pltpu.sync_copy(
                rowsi_v.at[pl.ds(0, 128)],
                xvt_hbm.at[pl.ds(pl.multiple_of(qbase + j * 128, 128), 128)])
            return carry

        lax.fori_loop(0, NQCH, q_gather_body, 0)

    return k


def _final_body(ms_ref, xvt_ref, d_ref,
                w2i_ref, b2i_ref, w2o_ref, b2o_ref,
                c1x_ref, c1i_ref, c1o_ref, cb1_ref, c2_ref, cb2_ref,
                s1a_ref, s1b_ref, s1c_ref, s1d_ref, sb1_ref, s2_ref, sb2_ref,
                out_ref):
    bb = d_ref.shape[0]

    ms = ms_ref[...] * (1.0 / K)
    m_in = jnp.dot(ms[:, :MSG_DIM], w2i_ref[...],
                   preferred_element_type=jnp.float32) + b2i_ref[...]
    m_out = jnp.dot(ms[:, MSG_DIM:], w2o_ref[...],
                    preferred_element_type=jnp.float32) + b2o_ref[...]

    pre1 = (jnp.dot(xvt_ref[...], c1x_ref[...],
                    preferred_element_type=jnp.float32)
            + jnp.dot(m_in, c1i_ref[...], preferred_element_type=jnp.float32)
            + jnp.dot(m_out, c1o_ref[...], preferred_element_type=jnp.float32)
            + cb1_ref[...])
    h1 = jnp.maximum(pre1, 0.0)
    h = jnp.maximum(
        jnp.dot(h1, c2_ref[...], preferred_element_type=jnp.float32)
        + cb2_ref[...], 0.0)
    h3 = h.reshape(bb, 3, H_DIM)
    d_norm = jnp.clip(d_ref[...], 0, T_VAL).astype(jnp.float32) / float(T_VAL)
    sc_pre = (jnp.dot(h3[:, 0, :], s1a_ref[...],
                      preferred_element_type=jnp.float32)
              + jnp.dot(h3[:, 1, :], s1b_ref[...],
                        preferred_element_type=jnp.float32)
              + jnp.dot(h3[:, 2, :], s1c_ref[...],
                        preferred_element_type=jnp.float32)
              + d_norm * s1d_ref[...] + sb1_ref[...])
    logits = (jnp.dot(jnp.maximum(sc_pre, 0.0), s2_ref[...],
                      preferred_element_type=jnp.float32) + sb2_ref[...])
    out_ref[...] = logits


def _final(msum, xvt, d2,
           w2i, b2i, w2o, b2o,
           c1x, c1i, c1o, cb1, c2, cb2, s1a, s1b, s1c, s1d, sb1, s2, sb2):
    Bb = 512
    NB = B // Bb
    bq = Bb * 3

    def full(shape):
        return pl.BlockSpec(shape, lambda i: tuple(0 for _ in shape))

    return pl.pallas_call(
        _final_body,
        grid=(NB,),
        in_specs=[
            pl.BlockSpec((bq, 2 * MSG_DIM), lambda i: (i, 0)),
            pl.BlockSpec((bq, F), lambda i: (i, 0)),
            pl.BlockSpec((Bb, 1), lambda i: (i, 0)),
            full((MSG_DIM, MSG_DIM)), full((MSG_DIM,)),
            full((MSG_DIM, MSG_DIM)), full((MSG_DIM,)),
            full((F, HIDDEN)), full((MSG_DIM, HIDDEN)), full((MSG_DIM, HIDDEN)),
            full((HIDDEN,)),
            full((HIDDEN, H_DIM)), full((H_DIM,)),
            full((H_DIM, HIDDEN)), full((H_DIM, HIDDEN)), full((H_DIM, HIDDEN)),
            full((1, HIDDEN)), full((HIDDEN,)),
            full((HIDDEN, 1)), full((1,)),
        ],
        out_specs=pl.BlockSpec((Bb, 1), lambda i: (i, 0)),
        out_shape=jax.ShapeDtypeStruct((B, 1), jnp.float32),
    )(msum, xvt, d2,
      w2i, b2i, w2o, b2o,
      c1x, c1i, c1o, cb1, c2, cb2, s1a, s1b, s1c, s1d, sb1, s2, sb2)


def kernel(x_tensor, nodes, t, in_u, in_tau, out_u, out_tau, d, delta_emb,
           phi_in_W1, phi_in_b1, phi_in_W2, phi_in_b2,
           phi_out_W1, phi_out_b1, phi_out_W2, phi_out_b2,
           comb_W1, comb_b1, comb_W2, comb_b2,
           score_W1, score_b1, score_W2, score_b2):
    x2 = x_tensor.reshape(NT, F)

    w1cat = jnp.concatenate([phi_in_W1[:F], phi_out_W1[:F]], axis=1)
    w1ecat = jnp.concatenate([phi_in_W1[F:], phi_out_W1[F:]], axis=1)
    b1cat = jnp.concatenate([phi_in_b1, phi_out_b1])

    pxcat, dtab = _precompute(x2, w1cat, delta_emb, w1ecat, b1cat)

    qt = jnp.broadcast_to(t[:, None], (B, 3)).reshape(MQ).astype(jnp.int32)
    msum, xvt = _sc_messages()(
        pxcat, x2, dtab,
        in_u.reshape(M).astype(jnp.int32), in_tau.reshape(M).astype(jnp.int32),
        out_u.reshape(M).astype(jnp.int32),
        out_tau.reshape(M).astype(jnp.int32),
        nodes.reshape(MQ).astype(jnp.int32), qt)

    d2 = d.reshape(B, 1).astype(jnp.int32)

    logits = _final(
        msum, xvt, d2,
        phi_in_W2, phi_in_b2, phi_out_W2, phi_out_b2,
        comb_W1[:F], comb_W1[F:F + MSG_DIM], comb_W1[F + MSG_DIM:], comb_b1,
        comb_W2, comb_b2,
        score_W1[:H_DIM], score_W1[H_DIM:2 * H_DIM],
        score_W1[2 * H_DIM:3 * H_DIM], score_W1[3 * H_DIM:], score_b1,
        score_W2, score_b2)
    return logits.reshape(B)


# speedup vs baseline: 21.8745x; 1.1319x over previous
import functools

import jax
import jax.numpy as jnp
from jax import lax
from jax.experimental import pallas as pl
from jax.experimental.pallas import tpu as pltpu
from jax.experimental.pallas import tpu_sc as plsc

N, T, F = 10000, 32, 128
B, K = 4096, 16
W = 32
T_VAL = 31
DELTA_DIM, MSG_DIM, HIDDEN, H_DIM = 16, 64, 128, 64
NT = N * T
M = B * 3 * K
MQ = B * 3

NC, NS, L = 2, 16, 16
NW = NC * NS

QW = MQ // NW
CQ = 4
NCH = QW // CQ
CHA = CQ * K
AW = QW * K
NQCH = QW // 128
TW = QW // 3


def _precompute_body(x_ref, w1i_ref, w1o_ref, demb_ref, b1i_ref, b1o_ref,
                     px_ref, dtab_ref):
    x = x_ref[...]
    w1i = w1i_ref[...]
    w1o = w1o_ref[...]
    pxi = jnp.dot(x, w1i[:F], preferred_element_type=jnp.float32)
    pxo = jnp.dot(x, w1o[:F], preferred_element_type=jnp.float32)
    px_ref[...] = jnp.concatenate([pxi, pxo], axis=1)
    demb = demb_ref[...]
    dti = jnp.dot(demb, w1i[F:], preferred_element_type=jnp.float32) \
        + b1i_ref[...]
    dto = jnp.dot(demb, w1o[F:], preferred_element_type=jnp.float32) \
        + b1o_ref[...]
    dtab_ref[...] = jnp.concatenate([dti, dto], axis=1)


def _precompute(x2, w1i, w1o, demb, b1i, b1o):
    RB = 3200
    grid = NT // RB

    def full(shape):
        return pl.BlockSpec(shape, lambda i: tuple(0 for _ in shape))

    return pl.pallas_call(
        _precompute_body,
        grid=(grid,),
        in_specs=[
            pl.BlockSpec((RB, F), lambda i: (i, 0)),
            full((F + DELTA_DIM, MSG_DIM)),
            full((F + DELTA_DIM, MSG_DIM)),
            full((W + 1, DELTA_DIM)),
            full((MSG_DIM,)),
            full((MSG_DIM,)),
        ],
        out_specs=[
            pl.BlockSpec((RB, 2 * MSG_DIM), lambda i: (i, 0)),
            full((W + 1, 2 * MSG_DIM)),
        ],
        out_shape=[
            jax.ShapeDtypeStruct((NT, 2 * MSG_DIM), jnp.float32),
            jax.ShapeDtypeStruct((W + 1, 2 * MSG_DIM), jnp.float32),
        ],
    )(x2, w1i, w1o, demb, b1i, b1o)


def _sc_messages():
    mesh = plsc.VectorSubcoreMesh(core_axis_name="c", subcore_axis_name="s")

    @functools.partial(
        pl.kernel,
        mesh=mesh,
        compiler_params=pltpu.CompilerParams(needs_layout_passes=False),
        out_type=[
            jax.ShapeDtypeStruct((MQ, 2 * MSG_DIM), jnp.float32),
            jax.ShapeDtypeStruct((MQ, F), jnp.float32),
        ],
        scratch_types=[
            pltpu.VMEM((AW,), jnp.int32),
            pltpu.VMEM((AW,), jnp.int32),
            pltpu.VMEM((QW,), jnp.int32),
            pltpu.VMEM((TW,), jnp.int32),
            pltpu.VMEM((NCH, CHA), jnp.int32),
            pltpu.VMEM((NCH, CHA), jnp.int32),
            pltpu.VMEM((NQCH, 128), jnp.int32),
            pltpu.VMEM((AW,), jnp.int32),
            pltpu.VMEM((AW,), jnp.int32),
            pltpu.VMEM((W + 1, 2 * MSG_DIM), jnp.float32),
            pltpu.VMEM((2, CHA, 2 * MSG_DIM), jnp.float32),
            pltpu.VMEM((2, CHA, 2 * MSG_DIM), jnp.float32),
            pltpu.VMEM((QW // 2, 2 * MSG_DIM), jnp.float32),
            pltpu.SemaphoreType.DMA,
            pltpu.SemaphoreType.DMA,
            pltpu.SemaphoreType.DMA,
            pltpu.SemaphoreType.DMA,
        ],
    )
    def k(px_hbm, x2_hbm, dtab_hbm, iu_hbm, itau_hbm, ou_hbm, otau_hbm,
          qn_hbm, t_hbm, msum_hbm, xvt_hbm,
          u_v, tau_v, qn_v, t_v, idxi_v, idxo_v, idxq_v, deli_v, delo_v,
          dtab_v, rowsi_v, rowso_v, ost_v, semA0, semA1, semB0, semB1):
        wid = lax.axis_index("s") * NC + lax.axis_index("c")
        qbase = pl.multiple_of(wid * QW, 128)
        abase = pl.multiple_of(wid * AW, 128)
        tbase = pl.multiple_of(wid * TW, 128)
        iota = lax.iota(jnp.int32, L)

        pltpu.sync_copy(dtab_hbm, dtab_v)
        pltpu.sync_copy(qn_hbm.at[pl.ds(qbase, QW)], qn_v)
        pltpu.sync_copy(t_hbm.at[pl.ds(tbase, TW)], t_v)

        def qidx_body(g, carry):
            s = pl.ds(g * L, L)
            qt16 = plsc.load_gather(t_v, [(g * L + iota) // 3])
            idxq_v[g // 8, pl.ds((g % 8) * L, L)] = qn_v[s] * T + qt16
            return carry

        lax.fori_loop(0, QW // L, qidx_body, 0)

        def prep_direction(uh, tauh, idx_v, del_v, sign):
            pltpu.sync_copy(uh.at[pl.ds(abase, AW)], u_v)
            pltpu.sync_copy(tauh.at[pl.ds(abase, AW)], tau_v)
            n_sub = CHA // L

            def body(g, carry):
                s = pl.ds(g * L, L)
                u16 = u_v[s]
                tau16 = tau_v[s]
                idx_v[g // n_sub, pl.ds((g % n_sub) * L, L)] = u16 * T + tau16
                tq = plsc.load_gather(t_v, [iota * 0 + (g // 3)])
                raw = (tq - tau16) * sign
                del_v[s] = jnp.minimum(jnp.maximum(raw, 0), W)
                return carry

            lax.fori_loop(0, AW // L, body, 0)

        prep_direction(iu_hbm, itau_hbm, idxi_v, deli_v, 1)
        prep_direction(ou_hbm, otau_hbm, idxo_v, delo_v, -1)

        semsA = [semA0, semA1]
        semsB = [semB0, semB1]

        def mk(j, p):
            pi = pltpu.make_async_copy(px_hbm.at[idxi_v.at[j]],
                                       rowsi_v.at[p], semsA[p])
            po = pltpu.make_async_copy(px_hbm.at[idxo_v.at[j]],
                                       rowso_v.at[p], semsB[p])
            return pi, po

        def start(j, p):
            pi, po = mk(j, p)
            pi.start()
            po.start()

        def compute(j, p, j0):
            pi, po = mk(j, p)
            pi.wait()
            po.wait()

            def q_body(qi, carry2):
                a0 = qi * K

                def reduce_dir(rows_v, del_v, colbase):
                    acc = [jnp.zeros((L,), jnp.float32) for _ in range(4)]
                    for kk in range(K):
                        a = a0 + kk
                        d16 = plsc.load_gather(
                            del_v, [iota * 0 + (j * CHA + a)])
                        for c in range(4):
                            dval = plsc.load_gather(
                                dtab_v, [d16, iota + (colbase + c * L)])
                            r = rows_v[p, a, pl.ds(colbase + c * L, L)]
                            acc[c] = acc[c] + jnp.maximum(r + dval, 0.0)
                    for c in range(4):
                        ost_v[(j - j0) * CQ + qi,
                              pl.ds(colbase + c * L, L)] = acc[c]

                reduce_dir(rowsi_v, deli_v, 0)
                reduce_dir(rowso_v, delo_v, MSG_DIM)
                return carry2

            lax.fori_loop(0, CQ, q_body, 0)

        NCH2 = NCH // 2
        for h in range(2):
            j0 = h * NCH2
            start(j0, 0)

            def chunk_body(jj, carry, j0=j0):
                j = j0 + jj * 2
                start(j + 1, 1)
                compute(j, 0, j0)
                start(j + 2, 0)
                compute(j + 1, 1, j0)
                return carry

            lax.fori_loop(0, NCH2 // 2 - 1, chunk_body, 0)
            start(j0 + NCH2 - 1, 1)
            compute(j0 + NCH2 - 2, 0, j0)
            compute(j0 + NCH2 - 1, 1, j0)
            pltpu.sync_copy(
                ost_v, msum_hbm.at[pl.ds(qbase + h * (QW // 2), QW // 2)])

        def q_gather_body(j, carry):
            cp = pltpu.make_async_copy(x2_hbm.at[idxq_v.at[j]],
                                       ost_v.at[pl.ds(0, 128)], semA0)
            cp.start()
            cp.wait()
            pltpu.sync_copy(
                ost_v.at[pl.ds(0, 128)],
                xvt_hbm.at[pl.ds(pl.multiple_of(qbase + j * 128, 128), 128)])
            return carry

        lax.fori_loop(0, NQCH, q_gather_body, 0)

    return k


def _final_body(ms_ref, xvt_ref, d_ref,
                w2i_ref, b2i_ref, w2o_ref, b2o_ref,
                c1_ref, cb1_ref, c2_ref, cb2_ref,
                s1_ref, sb1_ref, s2_ref, sb2_ref,
                out_ref):
    bb = d_ref.shape[0]

    ms = ms_ref[...] * (1.0 / K)
    m_in = jnp.dot(ms[:, :MSG_DIM], w2i_ref[...],
                   preferred_element_type=jnp.float32) + b2i_ref[...]
    m_out = jnp.dot(ms[:, MSG_DIM:], w2o_ref[...],
                    preferred_element_type=jnp.float32) + b2o_ref[...]

    c1 = c1_ref[...]
    pre1 = (jnp.dot(xvt_ref[...], c1[:F], preferred_element_type=jnp.float32)
            + jnp.dot(m_in, c1[F:F + MSG_DIM],
                      preferred_element_type=jnp.float32)
            + jnp.dot(m_out, c1[F + MSG_DIM:],
                      preferred_element_type=jnp.float32)
            + cb1_ref[...])
    h1 = jnp.maximum(pre1, 0.0)
    h = jnp.maximum(
        jnp.dot(h1, c2_ref[...], preferred_element_type=jnp.float32)
        + cb2_ref[...], 0.0)
    h3 = h.reshape(bb, 3, H_DIM)
    s1 = s1_ref[...]
    d_norm = jnp.clip(d_ref[...], 0, T_VAL).astype(jnp.float32) / float(T_VAL)
    sc_pre = (jnp.dot(h3[:, 0, :], s1[:H_DIM],
                      preferred_element_type=jnp.float32)
              + jnp.dot(h3[:, 1, :], s1[H_DIM:2 * H_DIM],
                        preferred_element_type=jnp.float32)
              + jnp.dot(h3[:, 2, :], s1[2 * H_DIM:3 * H_DIM],
                        preferred_element_type=jnp.float32)
              + d_norm * s1[3 * H_DIM:] + sb1_ref[...])
    logits = (jnp.dot(jnp.maximum(sc_pre, 0.0), s2_ref[...],
                      preferred_element_type=jnp.float32) + sb2_ref[...])
    out_ref[...] = logits


def _final(msum, xvt, d2, w2i, b2i, w2o, b2o,
           c1, cb1, c2, cb2, s1, sb1, s2, sb2):
    Bb = 512
    NB = B // Bb
    bq = Bb * 3

    def full(shape):
        return pl.BlockSpec(shape, lambda i: tuple(0 for _ in shape))

    return pl.pallas_call(
        _final_body,
        grid=(NB,),
        in_specs=[
            pl.BlockSpec((bq, 2 * MSG_DIM), lambda i: (i, 0)),
            pl.BlockSpec((bq, F), lambda i: (i, 0)),
            pl.BlockSpec((Bb, 1), lambda i: (i, 0)),
            full((MSG_DIM, MSG_DIM)), full((MSG_DIM,)),
            full((MSG_DIM, MSG_DIM)), full((MSG_DIM,)),
            full((F + 2 * MSG_DIM, HIDDEN)), full((HIDDEN,)),
            full((HIDDEN, H_DIM)), full((H_DIM,)),
            full((3 * H_DIM + 1, HIDDEN)), full((HIDDEN,)),
            full((HIDDEN, 1)), full((1,)),
        ],
        out_specs=pl.BlockSpec((Bb, 1), lambda i: (i, 0)),
        out_shape=jax.ShapeDtypeStruct((B, 1), jnp.float32),
    )(msum, xvt, d2, w2i, b2i, w2o, b2o,
      c1, cb1, c2, cb2, s1, sb1, s2, sb2)


def kernel(x_tensor, nodes, t, in_u, in_tau, out_u, out_tau, d, delta_emb,
           phi_in_W1, phi_in_b1, phi_in_W2, phi_in_b2,
           phi_out_W1, phi_out_b1, phi_out_W2, phi_out_b2,
           comb_W1, comb_b1, comb_W2, comb_b2,
           score_W1, score_b1, score_W2, score_b2):
    x2 = x_tensor.reshape(NT, F)

    pxcat, dtab = _precompute(x2, phi_in_W1, phi_out_W1, delta_emb,
                              phi_in_b1, phi_out_b1)

    msum, xvt = _sc_messages()(
        pxcat, x2, dtab,
        in_u.reshape(M).astype(jnp.int32), in_tau.reshape(M).astype(jnp.int32),
        out_u.reshape(M).astype(jnp.int32),
        out_tau.reshape(M).astype(jnp.int32),
        nodes.reshape(MQ).astype(jnp.int32), t.astype(jnp.int32))

    logits = _final(
        msum, xvt, d.reshape(B, 1).astype(jnp.int32),
        phi_in_W2, phi_in_b2, phi_out_W2, phi_out_b2,
        comb_W1, comb_b1, comb_W2, comb_b2,
        score_W1, score_b1, score_W2, score_b2)
    return logits.reshape(B)
